# Initial kernel scaffold; baseline (speedup 1.0000x reference)
import jax
import jax.numpy as jnp
from jax import lax
from jax.experimental import pallas as pl
from jax.experimental.pallas import tpu as pltpu
from jax.experimental.pallas import tpu_sc as plsc

B, T, N, F_IN, F_OUT, DEG = 32, 12, 325, 64, 64, 16
BT = B * T
NP = 328
NC, NS = 2, 16
NW = NC * NS
BT_PER = BT // NW
TCB = 8


def _tc_body(h_ref, wt_ref, a2_ref, hh_ref, s2_ref):
    hb = h_ref[...]
    h2 = hb.reshape(TCB * NP, F_IN)
    hh2 = jnp.dot(h2, wt_ref[...], preferred_element_type=jnp.float32)
    hh3 = hh2.reshape(TCB, NP, F_OUT)
    hh_ref[...] = hh3
    a2 = a2_ref[...].reshape(1, 1, F_OUT)
    s2_ref[...] = jnp.sum(hh3 * a2, axis=-1)


def _sc_body(hh_hbm, s2_hbm, adj_hbm, out_hbm, adj_v, hh_v, s2_v, out_v, alpha_v):
    cid = lax.axis_index("c")
    sid = lax.axis_index("s")
    wid = sid * NC + cid
    pltpu.sync_copy(adj_hbm, adj_v)

    def bt_body(k, carry):
        bt = wid * BT_PER + k
        pltpu.sync_copy(hh_hbm.at[bt], hh_v)
        pltpu.sync_copy(s2_hbm.at[bt], s2_v)

        def node_body(i, carry2):
            nbr = adj_v[i, :]
            svals = plsc.load_gather(s2_v, [nbr])
            m = jnp.max(svals)
            ex = jnp.exp(svals - m)
            alpha = ex / jnp.sum(ex)
            alpha_v[...] = alpha
            accs = [jnp.zeros((16,), jnp.float32) for _ in range(4)]
            for j in range(DEG):
                aj = alpha_v[j]
                ij = adj_v[i, j]
                for cb in range(4):
                    accs[cb] = accs[cb] + aj * hh_v[ij, pl.ds(cb * 16, 16)]
            for cb in range(4):
                out_v[i, pl.ds(cb * 16, 16)] = accs[cb]
            return carry2

        lax.fori_loop(0, N, node_body, 0)
        pltpu.sync_copy(out_v, out_hbm.at[bt])
        return carry

    lax.fori_loop(0, BT_PER, bt_body, 0)


def kernel(h, adj, W, a):
    hp = jnp.pad(h.reshape(BT, N, F_IN), ((0, 0), (0, NP - N), (0, 0)))
    wT = W.T
    a2 = a[F_OUT:].reshape(1, F_OUT)

    hh, s2 = pl.pallas_call(
        _tc_body,
        grid=(BT // TCB,),
        in_specs=[
            pl.BlockSpec((TCB, NP, F_IN), lambda i: (i, 0, 0)),
            pl.BlockSpec((F_IN, F_OUT), lambda i: (0, 0)),
            pl.BlockSpec((1, F_OUT), lambda i: (0, 0)),
        ],
        out_specs=[
            pl.BlockSpec((TCB, NP, F_OUT), lambda i: (i, 0, 0)),
            pl.BlockSpec((TCB, NP), lambda i: (i, 0)),
        ],
        out_shape=[
            jax.ShapeDtypeStruct((BT, NP, F_OUT), jnp.float32),
            jax.ShapeDtypeStruct((BT, NP), jnp.float32),
        ],
    )(hp, wT, a2)

    sc_fn = pl.kernel(
        _sc_body,
        out_type=jax.ShapeDtypeStruct((BT, NP, F_OUT), jnp.float32),
        mesh=plsc.VectorSubcoreMesh(core_axis_name="c", subcore_axis_name="s"),
        scratch_types=[
            pltpu.VMEM((N, DEG), jnp.int32),
            pltpu.VMEM((NP, F_OUT), jnp.float32),
            pltpu.VMEM((NP,), jnp.float32),
            pltpu.VMEM((NP, F_OUT), jnp.float32),
            pltpu.VMEM((DEG,), jnp.float32),
        ],
    )
    outp = sc_fn(hh, s2, adj)
    return outp[:, :N, :].reshape(B, T, N, F_OUT)


# speedup vs baseline: 5.2643x; 5.2643x over previous
import jax
import jax.numpy as jnp
from jax import lax
from jax.experimental import pallas as pl
from jax.experimental.pallas import tpu as pltpu
from jax.experimental.pallas import tpu_sc as plsc

B, T, N, F_IN, F_OUT, DEG = 32, 12, 325, 64, 64, 16
BT = B * T
NP = 328
NC, NS = 2, 16
NW = NC * NS
BT_PER = BT // NW
TCB = 8


def _tc_body(h_ref, wt_ref, a2_ref, hh_ref, s2_ref):
    hb = h_ref[...]
    h2 = hb.reshape(TCB * NP, F_IN)
    hh2 = jnp.dot(h2, wt_ref[...], preferred_element_type=jnp.float32)
    hh3 = hh2.reshape(TCB, NP, F_OUT)
    hh_ref[...] = hh3
    a2 = a2_ref[...].reshape(1, 1, F_OUT)
    s2_ref[...] = jnp.sum(hh3 * a2, axis=-1)


def _sc_body(hh_hbm, s2_hbm, adj_hbm, out_hbm, adj_v, hh_v, s2_v, out_v):
    cid = lax.axis_index("c")
    sid = lax.axis_index("s")
    wid = sid * NC + cid
    pltpu.sync_copy(adj_hbm, adj_v)

    def bt_body(k, carry):
        bt = wid * BT_PER + k
        pltpu.sync_copy(hh_hbm.at[bt], hh_v)
        pltpu.sync_copy(s2_hbm.at[bt], s2_v)

        def node_body(i, carry2):
            nbr = adj_v[i, :]
            svals = plsc.load_gather(s2_v, [nbr])
            m = jnp.max(svals)
            ex = jnp.exp(svals - m)
            alpha = ex / jnp.sum(ex)
            accs = [jnp.zeros((16,), jnp.float32) for _ in range(4)]
            for j in range(DEG):
                aj = alpha[j]
                ij = nbr[j]
                for cb in range(4):
                    accs[cb] = accs[cb] + aj * hh_v[ij, pl.ds(cb * 16, 16)]
            for cb in range(4):
                out_v[i, pl.ds(cb * 16, 16)] = accs[cb]
            return carry2

        lax.fori_loop(0, N, node_body, 0)
        pltpu.sync_copy(out_v, out_hbm.at[bt])
        return carry

    lax.fori_loop(0, BT_PER, bt_body, 0)


def kernel(h, adj, W, a):
    hp = jnp.pad(h.reshape(BT, N, F_IN), ((0, 0), (0, NP - N), (0, 0)))
    wT = W.T
    a2 = a[F_OUT:].reshape(1, F_OUT)

    hh, s2 = pl.pallas_call(
        _tc_body,
        grid=(BT // TCB,),
        in_specs=[
            pl.BlockSpec((TCB, NP, F_IN), lambda i: (i, 0, 0)),
            pl.BlockSpec((F_IN, F_OUT), lambda i: (0, 0)),
            pl.BlockSpec((1, F_OUT), lambda i: (0, 0)),
        ],
        out_specs=[
            pl.BlockSpec((TCB, NP, F_OUT), lambda i: (i, 0, 0)),
            pl.BlockSpec((TCB, NP), lambda i: (i, 0)),
        ],
        out_shape=[
            jax.ShapeDtypeStruct((BT, NP, F_OUT), jnp.float32),
            jax.ShapeDtypeStruct((BT, NP), jnp.float32),
        ],
    )(hp, wT, a2)

    sc_fn = pl.kernel(
        _sc_body,
        out_type=jax.ShapeDtypeStruct((BT, NP, F_OUT), jnp.float32),
        mesh=plsc.VectorSubcoreMesh(core_axis_name="c", subcore_axis_name="s",
                                    num_cores=NC, num_subcores=NS),
        compiler_params=pltpu.CompilerParams(needs_layout_passes=False),
        scratch_types=[
            pltpu.VMEM((N, DEG), jnp.int32),
            pltpu.VMEM((NP, F_OUT), jnp.float32),
            pltpu.VMEM((NP,), jnp.float32),
            pltpu.VMEM((NP, F_OUT), jnp.float32),
        ],
    )
    outp = sc_fn(hh, s2, adj)
    return outp[:, :N, :].reshape(B, T, N, F_OUT)
